# baseline (speedup 1.0000x reference)
import jax
import jax.numpy as jnp
from jax import lax
from jax.experimental import pallas as pl
from jax.experimental.pallas import tpu as pltpu
from jax.experimental.pallas import tpu_sc as plsc

N = 10000
E = 320000
D = 128
H = 64
G = 64

NP = 10240
RPT = NP // 16
NB = 2
SCH = 160
CH = SCH + NB
EPT = SCH * 128
EP = 16 * EPT

_mesh = plsc.VectorSubcoreMesh(core_axis_name="c", subcore_axis_name="s")
_sc_params = pltpu.CompilerParams(use_tc_tiling_on_sc=False,
                                  needs_layout_passes=False)

_F32 = jnp.float32
_BF16 = jnp.bfloat16


@pl.kernel(
    out_type=jax.ShapeDtypeStruct((2, NP), _F32),
    mesh=_mesh,
    scratch_types=[
        pltpu.VMEM((SCH, 128), jnp.int32),
        pltpu.VMEM((128,), _F32),
        pltpu.VMEM((RPT,), _F32),
        pltpu.VMEM_SHARED((NP,), _F32),
    ],
    compiler_params=_sc_params,
)
def _sc_deg(dst1_hbm, dst2_hbm, deg_out, dstbuf, ones, zbuf, deg_sh):
    c = lax.axis_index("c")
    s = lax.axis_index("s")

    @pl.loop(0, 128, step=16)
    def _(i):
        ones[pl.ds(i, 16)] = jnp.full((16,), 1.0, _F32)

    @pl.loop(0, RPT, step=16)
    def _(i):
        zbuf[pl.ds(i, 16)] = jnp.zeros((16,), _F32)

    pltpu.sync_copy(zbuf, deg_sh.at[pl.ds(s * RPT, RPT)])
    plsc.subcore_barrier()

    for g, dst_hbm in ((0, dst1_hbm), (1, dst2_hbm)):
        @pl.when(c == g)
        def _():
            pltpu.sync_copy(dst_hbm.at[s], dstbuf)

    @pl.loop(0, SCH)
    def _(j):
        pltpu.sync_copy(ones, deg_sh.at[dstbuf.at[j]], add=True)

    plsc.subcore_barrier()
    pltpu.sync_copy(deg_sh.at[pl.ds(s * RPT, RPT)],
                    deg_out.at[c, pl.ds(s * RPT, RPT)])


@pl.kernel(
    out_type=[
        jax.ShapeDtypeStruct((2, NP, H), _F32),
        jax.ShapeDtypeStruct((NP, H), _BF16),
        jax.ShapeDtypeStruct((NP, H), _BF16),
    ],
    mesh=_mesh,
    scratch_types=[
        pltpu.VMEM((CH, 128), jnp.int32),
        pltpu.VMEM((CH, 128), jnp.int32),
        pltpu.VMEM((64, H), _F32),
        pltpu.VMEM((RPT, H), _BF16),
        [pltpu.VMEM((128, H), _BF16) for _ in range(NB)],
        pltpu.VMEM((RPT,), _F32),
        pltpu.VMEM((RPT,), _F32),
        pltpu.VMEM((H,), _F32),
        [pltpu.SemaphoreType.DMA for _ in range(NB)],
        [pltpu.SemaphoreType.DMA for _ in range(NB)],
        pltpu.VMEM_SHARED((NP, H), _BF16),
    ],
    compiler_params=_sc_params,
)
def _sc_agg(h_hbm, deg_hbm, src1_hbm, dst1_hbm, src2_hbm, dst2_hbm, b_hbm,
            out_hbm, sc1_hbm, sc2_hbm,
            srcbuf, dstbuf, hbuf, sbuf, bufs, degbuf, dinvbuf, bbuf,
            gsems, ssems, acc_sh):
    c = lax.axis_index("c")
    s = lax.axis_index("s")

    pltpu.sync_copy(b_hbm, bbuf)
    pltpu.sync_copy(deg_hbm.at[c, pl.ds(s * RPT, RPT)], degbuf)

    @pl.loop(0, RPT, step=16)
    def _(i):
        d = degbuf[pl.ds(i, 16)] + 1.0
        bits = plsc.bitcast(d, jnp.int32)
        y = plsc.bitcast(jnp.int32(0x5F3759DF) - (bits >> 1), _F32)
        for _ in range(3):
            y = y * (1.5 - 0.5 * d * y * y)
        dinvbuf[pl.ds(i, 16)] = y

    def run(src_hbm, dst_hbm, sc_hbm, g):
        pltpu.sync_copy(src_hbm.at[s], srcbuf.at[pl.ds(0, SCH)])
        pltpu.sync_copy(dst_hbm.at[s], dstbuf.at[pl.ds(0, SCH)])
        for r in range(SCH, CH):
            @pl.loop(0, 128, step=16)
            def _(i):
                srcbuf[r, pl.ds(i, 16)] = jnp.zeros((16,), jnp.int32)
                dstbuf[r, pl.ds(i, 16)] = jnp.full((16,), N, jnp.int32)

        @pl.loop(0, RPT, step=64)
        def _(t):
            pltpu.sync_copy(h_hbm.at[pl.ds(g * NP + s * RPT + t, 64)], hbuf)
            for q in range(0, 64, 16):
                dvs = dinvbuf[pl.ds(t + q, 16)]
                for i in range(16):
                    dv = dvs[i]
                    for k in range(0, H, 32):
                        a = hbuf[q + i, pl.ds(k, 16)] * dv
                        b = hbuf[q + i, pl.ds(k + 16, 16)] * dv
                        sbuf[t + q + i, pl.ds(k, 32)] = plsc.pack(
                            a, b, format=plsc.PackFormat.INTERLEAVED)

        pltpu.sync_copy(sbuf, sc_hbm.at[pl.ds(s * RPT, RPT)])
        pltpu.sync_copy(sbuf, acc_sh.at[pl.ds(s * RPT, RPT)])
        plsc.subcore_barrier()

        def gather(j, k):
            pltpu.async_copy(sc_hbm.at[srcbuf.at[j]], bufs[k], gsems[k])

        def gwait(j, k):
            pltpu.make_async_copy(sc_hbm.at[srcbuf.at[j]], bufs[k],
                                  gsems[k]).wait()

        def scat(j, k):
            pltpu.async_copy(bufs[k], acc_sh.at[dstbuf.at[j]], ssems[k],
                             add=True)

        def swait(j, k):
            pltpu.make_async_copy(bufs[k], acc_sh.at[dstbuf.at[j]],
                                  ssems[k]).wait()

        for k in range(NB):
            gather(k, k)

        @pl.loop(0, SCH, step=NB)
        def _(j):
            for k in range(NB):
                gwait(j + k, k)
                scat(j + k, k)
                swait(j + k, k)
                gather(j + NB + k, k)

        for k in range(NB):
            gwait(SCH + k, k)

        plsc.subcore_barrier()

        pltpu.sync_copy(acc_sh.at[pl.ds(s * RPT, RPT)], sbuf)

        @pl.loop(0, RPT, step=64)
        def _(t):
            for q in range(0, 64, 16):
                dvs = dinvbuf[pl.ds(t + q, 16)]
                for i in range(16):
                    dv = dvs[i]
                    for k in range(0, H, 32):
                        av, bv = plsc.unpack(
                            sbuf[t + q + i, pl.ds(k, 32)],
                            format=plsc.PackFormat.INTERLEAVED,
                            preferred_element_type=_F32)
                        hbuf[q + i, pl.ds(k, 16)] = jnp.maximum(
                            av * dv + bbuf[pl.ds(k, 16)], 0.0)
                        hbuf[q + i, pl.ds(k + 16, 16)] = jnp.maximum(
                            bv * dv + bbuf[pl.ds(k + 16, 16)], 0.0)
            pltpu.sync_copy(hbuf, out_hbm.at[g, pl.ds(s * RPT + t, 64)])

    for g, (src_hbm, dst_hbm, sc_hbm) in (
            (0, (src1_hbm, dst1_hbm, sc1_hbm)),
            (1, (src2_hbm, dst2_hbm, sc2_hbm))):
        @pl.when(c == g)
        def _():
            run(src_hbm, dst_hbm, sc_hbm, g)


def _tc_h_body(x_ref, w_ref, o_ref):
    o_ref[...] = jnp.dot(x_ref[...], w_ref[...], preferred_element_type=_F32)


def _tc_h(xcat, w):
    blk = 1024
    return pl.pallas_call(
        _tc_h_body,
        grid=(2 * NP // blk,),
        in_specs=[
            pl.BlockSpec((blk, D), lambda i: (i, 0)),
            pl.BlockSpec((D, H), lambda i: (0, 0)),
        ],
        out_specs=pl.BlockSpec((blk, H), lambda i: (i, 0)),
        out_shape=jax.ShapeDtypeStruct((2 * NP, H), _F32),
    )(xcat, w)


def _tc_head_body(out_ref, batch_ref,
                  w1_ref, b1_ref, w2_ref, b2_ref, w3_ref, b3_ref,
                  w4_ref, b4_ref, o_ref):
    means = []
    for g in range(2):
        outg = out_ref[g]
        batchg = batch_ref[g]
        iot = lax.broadcasted_iota(jnp.int32, (G, NP), 0)
        oh = (iot == batchg).astype(_F32)
        sums = jnp.dot(oh, outg, preferred_element_type=_F32)
        cnts = jnp.sum(oh, axis=1, keepdims=True)
        means.append(sums / jnp.maximum(cnts, 1.0))
    z = jnp.concatenate(means, axis=1)
    z = jnp.maximum(jnp.dot(z, w1_ref[...],
                            preferred_element_type=_F32) + b1_ref[...], 0.0)
    z = jnp.maximum(jnp.dot(z, w2_ref[...],
                            preferred_element_type=_F32) + b2_ref[...], 0.0)
    z = jnp.maximum(jnp.dot(z, w3_ref[...],
                            preferred_element_type=_F32) + b3_ref[...], 0.0)
    z = jnp.dot(z, w4_ref[...], preferred_element_type=_F32) + b4_ref[...]
    o_ref[...] = 1.0 / (1.0 + jnp.exp(-z))


def _tc_head(outp, batch3, w1, b1, w2, b2, w3, b3, w4, b4):
    return pl.pallas_call(
        _tc_head_body,
        out_shape=jax.ShapeDtypeStruct((G, 1), _F32),
    )(outp, batch3, w1, b1, w2, b2, w3, b3, w4, b4)


def kernel(x_1, edge_index_1, x_1_batch, x_2, edge_index_2, x_2_batch,
           W_gcn, b_gcn, l1_w, l1_b, l2_w, l2_b, l3_w, l3_b, l4_w, l4_b):
    def prep(ei):
        src = jnp.pad(ei[0], (0, EP - E)).reshape(16, SCH, 128)
        dst = jnp.pad(ei[1], (0, EP - E),
                      constant_values=N).reshape(16, SCH, 128)
        return src, dst

    s1, d1 = prep(edge_index_1)
    s2, d2 = prep(edge_index_2)

    xcat = jnp.concatenate([
        jnp.pad(x_1, ((0, NP - N), (0, 0))),
        jnp.pad(x_2, ((0, NP - N), (0, 0))),
    ])

    deg = _sc_deg(d1, d2)
    h = _tc_h(xcat, W_gcn)
    outp, _, _ = _sc_agg(h, deg, s1, d1, s2, d2, b_gcn)

    batchcat = jnp.stack([
        jnp.pad(x_1_batch, (0, NP - N), constant_values=G),
        jnp.pad(x_2_batch, (0, NP - N), constant_values=G),
    ]).reshape(2, 1, NP)

    return _tc_head(
        outp, batchcat,
        l1_w, l1_b.reshape(1, 64),
        l2_w, l2_b.reshape(1, 32),
        l3_w, l3_b.reshape(1, 16),
        l4_w, l4_b.reshape(1, 1),
    )
